# baseline (speedup 1.0000x reference)
import functools

import jax
import jax.numpy as jnp
from jax import lax
from jax.experimental import pallas as pl
from jax.experimental.pallas import tpu as pltpu
from jax.experimental.pallas import tpu_sc as plsc


def _k1_body(xyz1_ref, xyz2t_ref, idx_ref, wgt_ref):
    b = pl.program_id(0)

    x1 = xyz1_ref[0]
    x2t = xyz2t_ref[0]
    t1 = x1.shape[0]
    n2 = x2t.shape[1]

    dot = jnp.dot(x1, x2t, preferred_element_type=jnp.float32)
    x1s = jnp.sum(x1 * x1, axis=1, keepdims=True)
    x2s = jnp.sum(x2t * x2t, axis=0, keepdims=True)
    sq = jnp.maximum((x1s + x2s) - 2.0 * dot, 1e-12)

    cols = jax.lax.broadcasted_iota(jnp.int32, sq.shape, 1)
    big = jnp.float32(3.0e38)

    m1 = jnp.min(sq, axis=1, keepdims=True)
    i1 = jnp.min(jnp.where(sq == m1, cols, n2), axis=1, keepdims=True)
    sqm = jnp.where(cols == i1, big, sq)
    m2 = jnp.min(sqm, axis=1, keepdims=True)
    i2 = jnp.min(jnp.where(sqm == m2, cols, n2), axis=1, keepdims=True)
    sqm = jnp.where(cols == i2, big, sqm)
    m3 = jnp.min(sqm, axis=1, keepdims=True)
    i3 = jnp.min(jnp.where(sqm == m3, cols, n2), axis=1, keepdims=True)

    r1 = 1.0 / (jnp.sqrt(m1) + 1e-8)
    r2 = 1.0 / (jnp.sqrt(m2) + 1e-8)
    r3 = 1.0 / (jnp.sqrt(m3) + 1e-8)
    s = r1 + r2 + r3

    off = b * n2
    z = jnp.zeros((t1, 5), jnp.int32)
    idx_ref[0] = jnp.concatenate([i1 + off, i2 + off, i3 + off, z], axis=1)
    zf = jnp.zeros((t1, 5), jnp.float32)
    wgt_ref[0] = jnp.concatenate([r1 / s, r2 / s, r3 / s, zf], axis=1)


def _make_sc_interp(M, C2, CH):
    NW = 32
    rows_per_w = M // NW
    nch = rows_per_w // CH
    mesh = plsc.VectorSubcoreMesh(core_axis_name="c", subcore_axis_name="s")

    @functools.partial(
        pl.kernel, mesh=mesh,
        out_type=jax.ShapeDtypeStruct((M, C2), jnp.float32),
        scratch_types=[
            pltpu.VMEM((CH * 8,), jnp.int32),
            pltpu.VMEM((CH * 8,), jnp.float32),
            pltpu.VMEM((CH * 8, C2), jnp.float32),
            pltpu.VMEM((CH, C2), jnp.float32),
            pltpu.SemaphoreType.DMA,
        ],
    )
    def sc_interp(feat2_hbm, idxf_hbm, wgtf_hbm, out_hbm,
                  idx_v, wgt_v, rows_v, out_v, sem):
        wid = lax.axis_index("s") * 2 + lax.axis_index("c")

        def chunk(ci, carry):
            base = (wid * rows_per_w // CH + ci) * CH
            pltpu.sync_copy(idxf_hbm.at[pl.ds(base * 8, CH * 8)], idx_v)
            pltpu.sync_copy(wgtf_hbm.at[pl.ds(base * 8, CH * 8)], wgt_v)
            pltpu.async_copy(feat2_hbm.at[idx_v], rows_v, sem).wait()

            dn = lax.GatherDimensionNumbers(
                offset_dims=(), collapsed_slice_dims=(0,),
                start_index_map=(0,))

            def lane_bcast(v, lane):
                return lax.gather(
                    v, jnp.full((16, 1), lane, jnp.int32), dn, (1,),
                    mode=lax.GatherScatterMode.PROMISE_IN_BOUNDS)

            for p in range(CH // 2):
                wload = wgt_v[pl.ds(p * 16, 16)]
                for half in range(2):
                    r = p * 2 + half
                    w1 = lane_bcast(wload, half * 8)
                    w2 = lane_bcast(wload, half * 8 + 1)
                    w3 = lane_bcast(wload, half * 8 + 2)
                    for cb in range(C2 // 16):
                        sl = pl.ds(cb * 16, 16)
                        out_v[r, sl] = (w1 * rows_v[8 * r, sl]
                                        + w2 * rows_v[8 * r + 1, sl]
                                        + w3 * rows_v[8 * r + 2, sl])

            pltpu.sync_copy(out_v, out_hbm.at[pl.ds(base, CH)])
            return carry

        lax.fori_loop(0, nch, chunk, 0)

    return sc_interp


def _k2a_body(feat1_ref, interp_ref, w0ta_ref, w0tb_ref, b0_ref,
              out_ref, stats_ref):
    x = (jnp.dot(feat1_ref[...], w0ta_ref[...],
                 preferred_element_type=jnp.float32)
         + jnp.dot(interp_ref[...], w0tb_ref[...],
                   preferred_element_type=jnp.float32)
         + b0_ref[...])
    out_ref[...] = x
    ps = jnp.sum(x, axis=0, keepdims=True)
    pss = jnp.sum(x * x, axis=0, keepdims=True)
    upd = jnp.concatenate([ps, pss, jnp.zeros((6, x.shape[1]), jnp.float32)],
                          axis=0)

    @pl.when(pl.program_id(0) == 0)
    def _():
        stats_ref[...] = jnp.zeros_like(stats_ref)

    stats_ref[...] += upd


def _k2b_body(x_ref, sc_ref, sh_ref, w1t_ref, b1_ref, out_ref, stats_ref):
    x = jnp.maximum(x_ref[...] * sc_ref[...] + sh_ref[...], 0.0)
    y = jnp.dot(x, w1t_ref[...], preferred_element_type=jnp.float32) + b1_ref[...]
    out_ref[...] = y
    ps = jnp.sum(y, axis=0, keepdims=True)
    pss = jnp.sum(y * y, axis=0, keepdims=True)
    upd = jnp.concatenate([ps, pss, jnp.zeros((6, y.shape[1]), jnp.float32)],
                          axis=0)

    @pl.when(pl.program_id(0) == 0)
    def _():
        stats_ref[...] = jnp.zeros_like(stats_ref)

    stats_ref[...] += upd


def _k3_body(x_ref, sc_ref, sh_ref, out_ref):
    out_ref[...] = jnp.maximum(x_ref[...] * sc_ref[...] + sh_ref[...], 0.0)


@jax.jit
def kernel(xyz1, xyz2, feat1, feat2, W0, b0, g0, be0, W1, b1, g1, be1):
    B, N1, _ = xyz1.shape
    N2 = xyz2.shape[1]
    C1 = feat1.shape[2]
    C2 = feat2.shape[2]
    H0 = W0.shape[0]
    H1 = W1.shape[0]
    M = B * N1

    T1 = min(1024, N1)
    xyz2t = jnp.swapaxes(xyz2, 1, 2)
    w0t = W0.T
    w1t = W1.T

    idx8, wgt8 = pl.pallas_call(
        _k1_body,
        grid=(B, N1 // T1),
        in_specs=[
            pl.BlockSpec((1, T1, 3), lambda b, i: (b, i, 0)),
            pl.BlockSpec((1, 3, N2), lambda b, i: (b, 0, 0)),
        ],
        out_specs=[
            pl.BlockSpec((1, T1, 8), lambda b, i: (b, i, 0)),
            pl.BlockSpec((1, T1, 8), lambda b, i: (b, i, 0)),
        ],
        out_shape=[
            jax.ShapeDtypeStruct((B, N1, 8), jnp.int32),
            jax.ShapeDtypeStruct((B, N1, 8), jnp.float32),
        ],
    )(xyz1, xyz2t)

    sc_interp = _make_sc_interp(M, C2, 64)
    interp = sc_interp(feat2.reshape(B * N2, C2),
                       idx8.reshape(M * 8),
                       wgt8.reshape(M * 8))

    T2 = min(2048, M)
    feat1_flat = feat1.reshape(M, C1)
    x1_pre, stats0 = pl.pallas_call(
        _k2a_body,
        grid=(M // T2,),
        in_specs=[
            pl.BlockSpec((T2, C1), lambda i: (i, 0)),
            pl.BlockSpec((T2, C2), lambda i: (i, 0)),
            pl.BlockSpec((C1, H0), lambda i: (0, 0)),
            pl.BlockSpec((C2, H0), lambda i: (0, 0)),
            pl.BlockSpec((1, H0), lambda i: (0, 0)),
        ],
        out_specs=[
            pl.BlockSpec((T2, H0), lambda i: (i, 0)),
            pl.BlockSpec((8, H0), lambda i: (0, 0)),
        ],
        out_shape=[
            jax.ShapeDtypeStruct((M, H0), jnp.float32),
            jax.ShapeDtypeStruct((8, H0), jnp.float32),
        ],
    )(feat1_flat, interp, w0t[:C1], w0t[C1:], b0.reshape(1, H0))

    mu0 = stats0[0] / M
    var0 = stats0[1] / M - mu0 * mu0
    sc0 = (g0 / jnp.sqrt(var0 + 1e-5)).reshape(1, H0)
    sh0 = (be0 - mu0 * g0 / jnp.sqrt(var0 + 1e-5)).reshape(1, H0)

    x2_pre, stats1 = pl.pallas_call(
        _k2b_body,
        grid=(M // T2,),
        in_specs=[
            pl.BlockSpec((T2, H0), lambda i: (i, 0)),
            pl.BlockSpec((1, H0), lambda i: (0, 0)),
            pl.BlockSpec((1, H0), lambda i: (0, 0)),
            pl.BlockSpec((H0, H1), lambda i: (0, 0)),
            pl.BlockSpec((1, H1), lambda i: (0, 0)),
        ],
        out_specs=[
            pl.BlockSpec((T2, H1), lambda i: (i, 0)),
            pl.BlockSpec((8, H1), lambda i: (0, 0)),
        ],
        out_shape=[
            jax.ShapeDtypeStruct((M, H1), jnp.float32),
            jax.ShapeDtypeStruct((8, H1), jnp.float32),
        ],
    )(x1_pre, sc0, sh0, w1t, b1.reshape(1, H1))

    mu1 = stats1[0] / M
    var1 = stats1[1] / M - mu1 * mu1
    sc1 = (g1 / jnp.sqrt(var1 + 1e-5)).reshape(1, H1)
    sh1 = (be1 - mu1 * g1 / jnp.sqrt(var1 + 1e-5)).reshape(1, H1)

    out = pl.pallas_call(
        _k3_body,
        grid=(M // T2,),
        in_specs=[
            pl.BlockSpec((T2, H1), lambda i: (i, 0)),
            pl.BlockSpec((1, H1), lambda i: (0, 0)),
            pl.BlockSpec((1, H1), lambda i: (0, 0)),
        ],
        out_specs=pl.BlockSpec((T2, H1), lambda i: (i, 0)),
        out_shape=jax.ShapeDtypeStruct((M, H1), jnp.float32),
    )(x2_pre, sc1, sh1)

    return out.reshape(B, N1, H1)


# speedup vs baseline: 10.8009x; 10.8009x over previous
import functools

import jax
import jax.numpy as jnp
from jax import lax
from jax.experimental import pallas as pl
from jax.experimental.pallas import tpu as pltpu
from jax.experimental.pallas import tpu_sc as plsc


def _k1_body(xyz1_ref, xyz2t_ref, idx_ref, wgt_ref):
    b = pl.program_id(0)

    x1 = xyz1_ref[0]
    x2t = xyz2t_ref[0]
    t1 = x1.shape[0]
    n2 = x2t.shape[1]

    dot = jnp.dot(x1, x2t, preferred_element_type=jnp.float32)
    x1s = jnp.sum(x1 * x1, axis=1, keepdims=True)
    x2s = jnp.sum(x2t * x2t, axis=0, keepdims=True)
    sq = jnp.maximum((x1s + x2s) - 2.0 * dot, 1e-12)

    cols = jax.lax.broadcasted_iota(jnp.int32, sq.shape, 1)
    big = jnp.float32(3.0e38)

    m1 = jnp.min(sq, axis=1, keepdims=True)
    i1 = jnp.min(jnp.where(sq == m1, cols, n2), axis=1, keepdims=True)
    sqm = jnp.where(cols == i1, big, sq)
    m2 = jnp.min(sqm, axis=1, keepdims=True)
    i2 = jnp.min(jnp.where(sqm == m2, cols, n2), axis=1, keepdims=True)
    sqm = jnp.where(cols == i2, big, sqm)
    m3 = jnp.min(sqm, axis=1, keepdims=True)
    i3 = jnp.min(jnp.where(sqm == m3, cols, n2), axis=1, keepdims=True)

    r1 = 1.0 / (jnp.sqrt(m1) + 1e-8)
    r2 = 1.0 / (jnp.sqrt(m2) + 1e-8)
    r3 = 1.0 / (jnp.sqrt(m3) + 1e-8)
    s = r1 + r2 + r3

    off = b * n2
    idx_ref[0] = jnp.concatenate(
        [i1 + off, i2 + off, i3 + off, i1 + off], axis=1)
    zf = jnp.zeros((t1, 1), jnp.float32)
    wgt_ref[0] = jnp.concatenate([r1 / s, r2 / s, r3 / s, zf], axis=1)


def _make_sc_interp(M, C2, CH):
    NW = 32
    rows_per_w = M // NW
    nch = rows_per_w // CH
    mesh = plsc.VectorSubcoreMesh(core_axis_name="c", subcore_axis_name="s")

    @functools.partial(
        pl.kernel, mesh=mesh,
        out_type=jax.ShapeDtypeStruct((M * 4, C2), jnp.float32),
        scratch_types=[
            pltpu.VMEM((CH * 4,), jnp.int32),
            pltpu.VMEM((CH * 4, C2), jnp.float32),
            pltpu.SemaphoreType.DMA,
        ],
    )
    def sc_interp(feat2_hbm, idxf_hbm, out_hbm, idx_v, rows_v, sem):
        wid = lax.axis_index("s") * 2 + lax.axis_index("c")

        def chunk(ci, carry):
            base = (wid * rows_per_w // CH + ci) * CH
            pltpu.sync_copy(idxf_hbm.at[pl.ds(base * 4, CH * 4)], idx_v)
            pltpu.async_copy(feat2_hbm.at[idx_v], rows_v, sem).wait()
            pltpu.sync_copy(rows_v, out_hbm.at[pl.ds(base * 4, CH * 4)])
            return carry

        lax.fori_loop(0, nch, chunk, 0)

    return sc_interp


def _k2a_body(feat1_ref, g_ref, wgt_ref, w0ta_ref, w0tb_ref, b0_ref,
              out_ref, stats_ref):
    g = g_ref[...]
    w = wgt_ref[...][:, :, None]
    interp = jnp.sum(g * w, axis=1)
    x = (jnp.dot(feat1_ref[...], w0ta_ref[...],
                 preferred_element_type=jnp.float32)
         + jnp.dot(interp, w0tb_ref[...],
                   preferred_element_type=jnp.float32)
         + b0_ref[...])
    out_ref[...] = x
    ps = jnp.sum(x, axis=0, keepdims=True)
    pss = jnp.sum(x * x, axis=0, keepdims=True)
    upd = jnp.concatenate([ps, pss, jnp.zeros((6, x.shape[1]), jnp.float32)],
                          axis=0)

    @pl.when(pl.program_id(0) == 0)
    def _():
        stats_ref[...] = jnp.zeros_like(stats_ref)

    stats_ref[...] += upd


def _k2b_body(x_ref, sc_ref, sh_ref, w1t_ref, b1_ref, out_ref, stats_ref):
    x = jnp.maximum(x_ref[...] * sc_ref[...] + sh_ref[...], 0.0)
    y = jnp.dot(x, w1t_ref[...], preferred_element_type=jnp.float32) + b1_ref[...]
    out_ref[...] = y
    ps = jnp.sum(y, axis=0, keepdims=True)
    pss = jnp.sum(y * y, axis=0, keepdims=True)
    upd = jnp.concatenate([ps, pss, jnp.zeros((6, y.shape[1]), jnp.float32)],
                          axis=0)

    @pl.when(pl.program_id(0) == 0)
    def _():
        stats_ref[...] = jnp.zeros_like(stats_ref)

    stats_ref[...] += upd


def _k3_body(x_ref, sc_ref, sh_ref, out_ref):
    out_ref[...] = jnp.maximum(x_ref[...] * sc_ref[...] + sh_ref[...], 0.0)


@jax.jit
def kernel(xyz1, xyz2, feat1, feat2, W0, b0, g0, be0, W1, b1, g1, be1):
    B, N1, _ = xyz1.shape
    N2 = xyz2.shape[1]
    C1 = feat1.shape[2]
    C2 = feat2.shape[2]
    H0 = W0.shape[0]
    H1 = W1.shape[0]
    M = B * N1

    T1 = min(1024, N1)
    xyz2t = jnp.swapaxes(xyz2, 1, 2)
    w0t = W0.T
    w1t = W1.T

    idx8, wgt8 = pl.pallas_call(
        _k1_body,
        grid=(B, N1 // T1),
        in_specs=[
            pl.BlockSpec((1, T1, 3), lambda b, i: (b, i, 0)),
            pl.BlockSpec((1, 3, N2), lambda b, i: (b, 0, 0)),
        ],
        out_specs=[
            pl.BlockSpec((1, T1, 4), lambda b, i: (b, i, 0)),
            pl.BlockSpec((1, T1, 4), lambda b, i: (b, i, 0)),
        ],
        out_shape=[
            jax.ShapeDtypeStruct((B, N1, 4), jnp.int32),
            jax.ShapeDtypeStruct((B, N1, 4), jnp.float32),
        ],
    )(xyz1, xyz2t)

    sc_interp = _make_sc_interp(M, C2, 128)
    gathered = sc_interp(feat2.reshape(B * N2, C2),
                         idx8.reshape(M * 4)).reshape(M, 4, C2)

    T2 = min(2048, M)
    feat1_flat = feat1.reshape(M, C1)
    x1_pre, stats0 = pl.pallas_call(
        _k2a_body,
        grid=(M // T2,),
        in_specs=[
            pl.BlockSpec((T2, C1), lambda i: (i, 0)),
            pl.BlockSpec((T2, 4, C2), lambda i: (i, 0, 0)),
            pl.BlockSpec((T2, 4), lambda i: (i, 0)),
            pl.BlockSpec((C1, H0), lambda i: (0, 0)),
            pl.BlockSpec((C2, H0), lambda i: (0, 0)),
            pl.BlockSpec((1, H0), lambda i: (0, 0)),
        ],
        out_specs=[
            pl.BlockSpec((T2, H0), lambda i: (i, 0)),
            pl.BlockSpec((8, H0), lambda i: (0, 0)),
        ],
        out_shape=[
            jax.ShapeDtypeStruct((M, H0), jnp.float32),
            jax.ShapeDtypeStruct((8, H0), jnp.float32),
        ],
    )(feat1_flat, gathered, wgt8.reshape(M, 4), w0t[:C1], w0t[C1:],
      b0.reshape(1, H0))

    mu0 = stats0[0] / M
    var0 = stats0[1] / M - mu0 * mu0
    sc0 = (g0 / jnp.sqrt(var0 + 1e-5)).reshape(1, H0)
    sh0 = (be0 - mu0 * g0 / jnp.sqrt(var0 + 1e-5)).reshape(1, H0)

    x2_pre, stats1 = pl.pallas_call(
        _k2b_body,
        grid=(M // T2,),
        in_specs=[
            pl.BlockSpec((T2, H0), lambda i: (i, 0)),
            pl.BlockSpec((1, H0), lambda i: (0, 0)),
            pl.BlockSpec((1, H0), lambda i: (0, 0)),
            pl.BlockSpec((H0, H1), lambda i: (0, 0)),
            pl.BlockSpec((1, H1), lambda i: (0, 0)),
        ],
        out_specs=[
            pl.BlockSpec((T2, H1), lambda i: (i, 0)),
            pl.BlockSpec((8, H1), lambda i: (0, 0)),
        ],
        out_shape=[
            jax.ShapeDtypeStruct((M, H1), jnp.float32),
            jax.ShapeDtypeStruct((8, H1), jnp.float32),
        ],
    )(x1_pre, sc0, sh0, w1t, b1.reshape(1, H1))

    mu1 = stats1[0] / M
    var1 = stats1[1] / M - mu1 * mu1
    sc1 = (g1 / jnp.sqrt(var1 + 1e-5)).reshape(1, H1)
    sh1 = (be1 - mu1 * g1 / jnp.sqrt(var1 + 1e-5)).reshape(1, H1)

    out = pl.pallas_call(
        _k3_body,
        grid=(M // T2,),
        in_specs=[
            pl.BlockSpec((T2, H1), lambda i: (i, 0)),
            pl.BlockSpec((1, H1), lambda i: (0, 0)),
            pl.BlockSpec((1, H1), lambda i: (0, 0)),
        ],
        out_specs=pl.BlockSpec((T2, H1), lambda i: (i, 0)),
        out_shape=jax.ShapeDtypeStruct((M, H1), jnp.float32),
    )(x2_pre, sc1, sh1)

    return out.reshape(B, N1, H1)


# speedup vs baseline: 10.9263x; 1.0116x over previous
import functools

import jax
import jax.numpy as jnp
from jax import lax
from jax.experimental import pallas as pl
from jax.experimental.pallas import tpu as pltpu
from jax.experimental.pallas import tpu_sc as plsc


def _k1_body(xyz1_ref, xyz2t_ref, idx_ref, wgt_ref):
    b = pl.program_id(0)

    x1 = xyz1_ref[0]
    x2t = xyz2t_ref[0]
    t1 = x1.shape[0]
    n2 = x2t.shape[1]

    dot = jnp.dot(x1, x2t, preferred_element_type=jnp.float32)
    x1s = jnp.sum(x1 * x1, axis=1, keepdims=True)
    x2s = jnp.sum(x2t * x2t, axis=0, keepdims=True)
    sq = jnp.maximum((x1s + x2s) - 2.0 * dot, 1e-12)

    cols = jax.lax.broadcasted_iota(jnp.int32, sq.shape, 1)
    big = jnp.float32(3.0e38)

    m1 = jnp.min(sq, axis=1, keepdims=True)
    i1 = jnp.min(jnp.where(sq == m1, cols, n2), axis=1, keepdims=True)
    sqm = jnp.where(cols == i1, big, sq)
    m2 = jnp.min(sqm, axis=1, keepdims=True)
    i2 = jnp.min(jnp.where(sqm == m2, cols, n2), axis=1, keepdims=True)
    sqm = jnp.where(cols == i2, big, sqm)
    m3 = jnp.min(sqm, axis=1, keepdims=True)
    i3 = jnp.min(jnp.where(sqm == m3, cols, n2), axis=1, keepdims=True)

    r1 = 1.0 / (jnp.sqrt(m1) + 1e-8)
    r2 = 1.0 / (jnp.sqrt(m2) + 1e-8)
    r3 = 1.0 / (jnp.sqrt(m3) + 1e-8)
    s = r1 + r2 + r3

    off = b * n2
    idx_ref[0] = jnp.concatenate(
        [i1 + off, i2 + off, i3 + off, i1 + off], axis=1)
    zf = jnp.zeros((t1, 1), jnp.float32)
    wgt_ref[0] = jnp.concatenate([r1 / s, r2 / s, r3 / s, zf], axis=1)


def _make_sc_interp(M, C2, CH):
    NW = 32
    rows_per_w = M // NW
    nch = rows_per_w // CH
    mesh = plsc.VectorSubcoreMesh(core_axis_name="c", subcore_axis_name="s")

    @functools.partial(
        pl.kernel, mesh=mesh,
        out_type=jax.ShapeDtypeStruct((M * 4, C2), jnp.float32),
        scratch_types=[
            pltpu.VMEM((CH * 4,), jnp.int32),
            pltpu.VMEM((CH * 4,), jnp.int32),
            pltpu.VMEM((CH * 4, C2), jnp.float32),
            pltpu.VMEM((CH * 4, C2), jnp.float32),
            pltpu.SemaphoreType.DMA,
            pltpu.SemaphoreType.DMA,
        ],
    )
    def sc_interp(feat2_hbm, idxf_hbm, out_hbm,
                  idx_v0, idx_v1, rows_v0, rows_v1, sem0, sem1):
        wid = lax.axis_index("s") * 2 + lax.axis_index("c")
        idx_vs = (idx_v0, idx_v1)
        rows_vs = (rows_v0, rows_v1)
        sems = (sem0, sem1)

        base0 = (wid * rows_per_w // CH) * CH
        pltpu.sync_copy(idxf_hbm.at[pl.ds(base0 * 4, CH * 4)], idx_v0)
        pltpu.async_copy(feat2_hbm.at[idx_v0], rows_v0, sem0)

        def pair(p, carry):
            for b in range(2):
                ci = 2 * p + b
                nb = 1 - b

                @pl.when(ci + 1 < nch)
                def _():
                    basen = (wid * rows_per_w // CH + ci + 1) * CH
                    pltpu.sync_copy(
                        idxf_hbm.at[pl.ds(basen * 4, CH * 4)], idx_vs[nb])
                    pltpu.async_copy(
                        feat2_hbm.at[idx_vs[nb]], rows_vs[nb], sems[nb])

                base = (wid * rows_per_w // CH + ci) * CH
                pltpu.make_async_copy(
                    feat2_hbm.at[idx_vs[b]], rows_vs[b], sems[b]).wait()
                pltpu.sync_copy(
                    rows_vs[b], out_hbm.at[pl.ds(base * 4, CH * 4)])
            return carry

        lax.fori_loop(0, nch // 2, pair, 0)

    return sc_interp


def _k2a_body(feat1_ref, g_ref, wgt_ref, w0ta_ref, w0tb_ref, b0_ref,
              out_ref, stats_ref):
    g = g_ref[...]
    w = wgt_ref[...][:, :, None]
    interp = jnp.sum(g * w, axis=1)
    x = (jnp.dot(feat1_ref[...], w0ta_ref[...],
                 preferred_element_type=jnp.float32)
         + jnp.dot(interp, w0tb_ref[...],
                   preferred_element_type=jnp.float32)
         + b0_ref[...])
    out_ref[...] = x
    ps = jnp.sum(x, axis=0, keepdims=True)
    pss = jnp.sum(x * x, axis=0, keepdims=True)
    upd = jnp.concatenate([ps, pss, jnp.zeros((6, x.shape[1]), jnp.float32)],
                          axis=0)

    @pl.when(pl.program_id(0) == 0)
    def _():
        stats_ref[...] = jnp.zeros_like(stats_ref)

    stats_ref[...] += upd


def _k2b_body(x_ref, sc_ref, sh_ref, w1t_ref, b1_ref, out_ref, stats_ref):
    x = jnp.maximum(x_ref[...] * sc_ref[...] + sh_ref[...], 0.0)
    y = jnp.dot(x, w1t_ref[...], preferred_element_type=jnp.float32) + b1_ref[...]
    out_ref[...] = y
    ps = jnp.sum(y, axis=0, keepdims=True)
    pss = jnp.sum(y * y, axis=0, keepdims=True)
    upd = jnp.concatenate([ps, pss, jnp.zeros((6, y.shape[1]), jnp.float32)],
                          axis=0)

    @pl.when(pl.program_id(0) == 0)
    def _():
        stats_ref[...] = jnp.zeros_like(stats_ref)

    stats_ref[...] += upd


def _k3_body(x_ref, sc_ref, sh_ref, out_ref):
    out_ref[...] = jnp.maximum(x_ref[...] * sc_ref[...] + sh_ref[...], 0.0)


@jax.jit
def kernel(xyz1, xyz2, feat1, feat2, W0, b0, g0, be0, W1, b1, g1, be1):
    B, N1, _ = xyz1.shape
    N2 = xyz2.shape[1]
    C1 = feat1.shape[2]
    C2 = feat2.shape[2]
    H0 = W0.shape[0]
    H1 = W1.shape[0]
    M = B * N1

    T1 = min(1024, N1)
    xyz2t = jnp.swapaxes(xyz2, 1, 2)
    w0t = W0.T
    w1t = W1.T

    idx8, wgt8 = pl.pallas_call(
        _k1_body,
        grid=(B, N1 // T1),
        in_specs=[
            pl.BlockSpec((1, T1, 3), lambda b, i: (b, i, 0)),
            pl.BlockSpec((1, 3, N2), lambda b, i: (b, 0, 0)),
        ],
        out_specs=[
            pl.BlockSpec((1, T1, 4), lambda b, i: (b, i, 0)),
            pl.BlockSpec((1, T1, 4), lambda b, i: (b, i, 0)),
        ],
        out_shape=[
            jax.ShapeDtypeStruct((B, N1, 4), jnp.int32),
            jax.ShapeDtypeStruct((B, N1, 4), jnp.float32),
        ],
    )(xyz1, xyz2t)

    sc_interp = _make_sc_interp(M, C2, 64)
    gathered = sc_interp(feat2.reshape(B * N2, C2),
                         idx8.reshape(M * 4)).reshape(M, 4, C2)

    T2 = min(2048, M)
    feat1_flat = feat1.reshape(M, C1)
    x1_pre, stats0 = pl.pallas_call(
        _k2a_body,
        grid=(M // T2,),
        in_specs=[
            pl.BlockSpec((T2, C1), lambda i: (i, 0)),
            pl.BlockSpec((T2, 4, C2), lambda i: (i, 0, 0)),
            pl.BlockSpec((T2, 4), lambda i: (i, 0)),
            pl.BlockSpec((C1, H0), lambda i: (0, 0)),
            pl.BlockSpec((C2, H0), lambda i: (0, 0)),
            pl.BlockSpec((1, H0), lambda i: (0, 0)),
        ],
        out_specs=[
            pl.BlockSpec((T2, H0), lambda i: (i, 0)),
            pl.BlockSpec((8, H0), lambda i: (0, 0)),
        ],
        out_shape=[
            jax.ShapeDtypeStruct((M, H0), jnp.float32),
            jax.ShapeDtypeStruct((8, H0), jnp.float32),
        ],
    )(feat1_flat, gathered, wgt8.reshape(M, 4), w0t[:C1], w0t[C1:],
      b0.reshape(1, H0))

    mu0 = stats0[0] / M
    var0 = stats0[1] / M - mu0 * mu0
    sc0 = (g0 / jnp.sqrt(var0 + 1e-5)).reshape(1, H0)
    sh0 = (be0 - mu0 * g0 / jnp.sqrt(var0 + 1e-5)).reshape(1, H0)

    x2_pre, stats1 = pl.pallas_call(
        _k2b_body,
        grid=(M // T2,),
        in_specs=[
            pl.BlockSpec((T2, H0), lambda i: (i, 0)),
            pl.BlockSpec((1, H0), lambda i: (0, 0)),
            pl.BlockSpec((1, H0), lambda i: (0, 0)),
            pl.BlockSpec((H0, H1), lambda i: (0, 0)),
            pl.BlockSpec((1, H1), lambda i: (0, 0)),
        ],
        out_specs=[
            pl.BlockSpec((T2, H1), lambda i: (i, 0)),
            pl.BlockSpec((8, H1), lambda i: (0, 0)),
        ],
        out_shape=[
            jax.ShapeDtypeStruct((M, H1), jnp.float32),
            jax.ShapeDtypeStruct((8, H1), jnp.float32),
        ],
    )(x1_pre, sc0, sh0, w1t, b1.reshape(1, H1))

    mu1 = stats1[0] / M
    var1 = stats1[1] / M - mu1 * mu1
    sc1 = (g1 / jnp.sqrt(var1 + 1e-5)).reshape(1, H1)
    sh1 = (be1 - mu1 * g1 / jnp.sqrt(var1 + 1e-5)).reshape(1, H1)

    out = pl.pallas_call(
        _k3_body,
        grid=(M // T2,),
        in_specs=[
            pl.BlockSpec((T2, H1), lambda i: (i, 0)),
            pl.BlockSpec((1, H1), lambda i: (0, 0)),
            pl.BlockSpec((1, H1), lambda i: (0, 0)),
        ],
        out_specs=pl.BlockSpec((T2, H1), lambda i: (i, 0)),
        out_shape=jax.ShapeDtypeStruct((M, H1), jnp.float32),
    )(x2_pre, sc1, sh1)

    return out.reshape(B, N1, H1)
